# Initial kernel scaffold; baseline (speedup 1.0000x reference)
import jax
import jax.numpy as jnp
from jax.experimental import pallas as pl


def kernel(x, emb_table, linear_weights):
    raise NotImplementedError("write your pallas kernel here")


# speedup vs baseline: 1.8758x; 1.8758x over previous
import jax
import jax.numpy as jnp
from jax.experimental import pallas as pl

_B = 1024
_F = 100
_D = 16
_V = 100
_TI = 128


def _stage_a(x_ref, xt_ref, emb_ref, w_ref, ind_ref, cross_ref, linexp_ref):
    xb = x_ref[...]
    iota = jax.lax.broadcasted_iota(jnp.int32, (1, 1, _V), 2)
    eq = (xb[:, :, None] == iota).astype(jnp.float32)
    cmat = jnp.sum(eq, axis=1)
    eb = emb_ref[...]
    se = jnp.dot(cmat, eb, preferred_element_type=jnp.float32, precision=jax.lax.Precision.HIGHEST)
    se2 = jnp.dot(cmat, eb * eb, preferred_element_type=jnp.float32, precision=jax.lax.Precision.HIGHEST)
    cross_ref[...] = 0.5 * (se * se - se2)
    lin_row = jnp.dot(w_ref[...], xt_ref[...],
                      preferred_element_type=jnp.float32, precision=jax.lax.Precision.HIGHEST)
    linexp_ref[...] = jnp.dot(lin_row, ind_ref[...],
                              preferred_element_type=jnp.float32, precision=jax.lax.Precision.HIGHEST)


def _stage_b(cross_ref, linexp_ref, t_ref, out_ref):
    expanded = jnp.dot(cross_ref[...], t_ref[...],
                       preferred_element_type=jnp.float32, precision=jax.lax.Precision.HIGHEST)
    out_ref[...] = jax.nn.sigmoid(expanded + linexp_ref[...])


def kernel(x, emb_table, linear_weights):
    w_row = linear_weights.reshape(1, _F)
    xt = x.astype(jnp.float32).T
    ind = jnp.repeat(jnp.eye(_TI, dtype=jnp.float32), _D, axis=1)
    n_i = _B // _TI

    cross, linexp = pl.pallas_call(
        _stage_a,
        grid=(n_i,),
        in_specs=[
            pl.BlockSpec((_TI, _F), lambda i: (i, 0)),
            pl.BlockSpec((_F, _TI), lambda i: (0, i)),
            pl.BlockSpec((_V, _D), lambda i: (0, 0)),
            pl.BlockSpec((1, _F), lambda i: (0, 0)),
            pl.BlockSpec((_TI, _TI * _D), lambda i: (0, 0)),
        ],
        out_specs=[
            pl.BlockSpec((_TI, _D), lambda i: (i, 0)),
            pl.BlockSpec((1, _TI * _D), lambda i: (0, i)),
        ],
        out_shape=[
            jax.ShapeDtypeStruct((_B, _D), jnp.float32),
            jax.ShapeDtypeStruct((1, _B * _D), jnp.float32),
        ],
    )(x, xt, emb_table, w_row, ind)

    t_mat = jnp.tile(jnp.eye(_D, dtype=jnp.float32), (1, _B))

    out2 = pl.pallas_call(
        _stage_b,
        grid=(n_i,),
        in_specs=[
            pl.BlockSpec((_TI, _D), lambda i: (i, 0)),
            pl.BlockSpec((1, _B * _D), lambda i: (0, 0)),
            pl.BlockSpec((_D, _B * _D), lambda i: (0, 0)),
        ],
        out_specs=pl.BlockSpec((_TI, _B * _D), lambda i: (i, 0)),
        out_shape=jax.ShapeDtypeStruct((_B, _B * _D), jnp.float32),
    )(cross, linexp, t_mat)

    return out2.reshape(_B, _B, _D)


# speedup vs baseline: 8.2212x; 4.3827x over previous
import jax
import jax.numpy as jnp
from jax.experimental import pallas as pl

_B = 1024
_F = 100
_D = 16
_V = 100
_TI = 128
_TR = _TI * _D


def _fm_kernel(x_ref, xt_ref, wcol_ref, m1_ref, eg_ref, out_ref):
    xb = x_ref[...]
    iota = jax.lax.broadcasted_iota(jnp.int32, (1, 1, _V), 2)
    eq = (xb[:, :, None] == iota).astype(jnp.float32)
    cmat = jnp.sum(eq, axis=1)
    cexp = jnp.dot(m1_ref[...], cmat,
                   preferred_element_type=jnp.float32)
    eg = eg_ref[...]
    se = jnp.sum(cexp * eg, axis=1, keepdims=True)
    se2 = jnp.sum(cexp * (eg * eg), axis=1, keepdims=True)
    cross_col = 0.5 * (se * se - se2)
    lin_row = jnp.sum(wcol_ref[...] * xt_ref[...], axis=0, keepdims=True)
    t = cross_col + lin_row
    out_ref[...] = 0.5 * jnp.tanh(0.5 * t) + 0.5


def kernel(x, emb_table, linear_weights):
    n_i = _B // _TI
    xt = x.astype(jnp.float32).T
    wcol = linear_weights.reshape(_F, 1)
    m1 = jnp.repeat(jnp.eye(_TI, dtype=jnp.float32), _D, axis=0)
    eg = jnp.tile(emb_table.T, (_TI, 1))

    out2 = pl.pallas_call(
        _fm_kernel,
        grid=(n_i,),
        in_specs=[
            pl.BlockSpec((_TI, _F), lambda i: (i, 0)),
            pl.BlockSpec((_F, _B), lambda i: (0, 0)),
            pl.BlockSpec((_F, 1), lambda i: (0, 0)),
            pl.BlockSpec((_TR, _TI), lambda i: (0, 0)),
            pl.BlockSpec((_TR, _V), lambda i: (0, 0)),
        ],
        out_specs=pl.BlockSpec((_TR, _B), lambda i: (i, 0)),
        out_shape=jax.ShapeDtypeStruct((_B * _D, _B), jnp.float32),
    )(x, xt, wcol, m1, eg)

    return out2.reshape(_B, _D, _B).transpose(0, 2, 1)
